# initial kernel scaffold (unmeasured)
import jax
import jax.numpy as jnp
from jax import lax
from jax.experimental import pallas as pl
from jax.experimental.pallas import tpu as pltpu

SCALE = 128 ** -0.5


def _flash_partial_body(q_ref, k_ref, v_ref, n_ref, l_ref):
    q = q_ref[0, :, 0, :].astype(jnp.bfloat16)
    k = k_ref[0, :, 0, :].astype(jnp.bfloat16)
    v = v_ref[0, :, 0, :].astype(jnp.bfloat16)
    s = lax.dot_general(
        q, k, (((1,), (1,)), ((), ())), preferred_element_type=jnp.float32
    )
    p = jnp.exp(s * SCALE)
    l_ref[0, :, 0] = jnp.sum(p, axis=1)
    n = lax.dot_general(
        p.astype(jnp.bfloat16), v, (((1,), (0,)), ((), ())),
        preferred_element_type=jnp.float32,
    )
    n_ref[0, :, 0, :] = n.astype(jnp.bfloat16)


def _combine_body(n_ref, l_ref, o_ref, ncomm_ref, lcomm_ref, send_sems, recv_sems):
    my_x = lax.axis_index("x")
    my_y = lax.axis_index("y")
    nbr = (1 - my_x, my_y)

    bar = pltpu.get_barrier_semaphore()
    pl.semaphore_signal(
        bar, inc=1, device_id=nbr, device_id_type=pl.DeviceIdType.MESH
    )
    pl.semaphore_wait(bar, 1)

    cp_n = pltpu.make_async_remote_copy(
        src_ref=n_ref, dst_ref=ncomm_ref,
        send_sem=send_sems.at[0], recv_sem=recv_sems.at[0],
        device_id=nbr, device_id_type=pl.DeviceIdType.MESH,
    )
    cp_l = pltpu.make_async_remote_copy(
        src_ref=l_ref, dst_ref=lcomm_ref,
        send_sem=send_sems.at[1], recv_sem=recv_sems.at[1],
        device_id=nbr, device_id_type=pl.DeviceIdType.MESH,
    )
    cp_n.start()
    cp_l.start()
    cp_n.wait()
    cp_l.wait()

    n_sum = n_ref[...].astype(jnp.float32) + ncomm_ref[...].astype(jnp.float32)
    l_sum = l_ref[...] + lcomm_ref[...]
    o_ref[...] = n_sum / l_sum[..., None]


def kernel(Q, K, V):
    b, q, h, d = Q.shape
    kv = K.shape[1]

    n_part, l_part = pl.pallas_call(
        _flash_partial_body,
        grid=(b, h),
        in_specs=[
            pl.BlockSpec((1, q, 1, d), lambda i, j: (i, 0, j, 0)),
            pl.BlockSpec((1, kv, 1, d), lambda i, j: (i, 0, j, 0)),
            pl.BlockSpec((1, kv, 1, d), lambda i, j: (i, 0, j, 0)),
        ],
        out_specs=[
            pl.BlockSpec((1, q, 1, d), lambda i, j: (i, 0, j, 0)),
            pl.BlockSpec((1, q, 1), lambda i, j: (i, 0, j)),
        ],
        out_shape=[
            jax.ShapeDtypeStruct((b, q, h, d), jnp.bfloat16),
            jax.ShapeDtypeStruct((b, q, h), jnp.float32),
        ],
    )(Q, K, V)

    return pl.pallas_call(
        _combine_body,
        in_specs=[
            pl.BlockSpec(memory_space=pltpu.VMEM),
            pl.BlockSpec(memory_space=pltpu.VMEM),
        ],
        out_specs=pl.BlockSpec(memory_space=pltpu.VMEM),
        out_shape=jax.ShapeDtypeStruct((b, q, h, d), jnp.float32),
        scratch_shapes=[
            pltpu.VMEM((b, q, h, d), jnp.bfloat16),
            pltpu.VMEM((b, q, h), jnp.float32),
            pltpu.SemaphoreType.DMA((2,)),
            pltpu.SemaphoreType.DMA((2,)),
        ],
        compiler_params=pltpu.CompilerParams(collective_id=0),
    )(n_part, l_part)


# baseline (device time: 149700 ns/iter reference)
import jax
import jax.numpy as jnp
from jax import lax
from jax.experimental import pallas as pl
from jax.experimental.pallas import tpu as pltpu

SCALE = 128 ** -0.5


def _flash_partial_body(q_ref, k_ref, v_ref, n_ref, l_ref):
    n_heads = q_ref.shape[2]
    for hh in range(n_heads):
        q = q_ref[0, :, hh, :].astype(jnp.bfloat16)
        k = k_ref[0, :, hh, :].astype(jnp.bfloat16)
        v = v_ref[0, :, hh, :].astype(jnp.bfloat16)
        s = lax.dot_general(
            q, k, (((1,), (1,)), ((), ())), preferred_element_type=jnp.float32
        )
        p = jnp.exp(s * SCALE)
        l_ref[0, :, hh] = jnp.sum(p, axis=1)
        n = lax.dot_general(
            p.astype(jnp.bfloat16), v, (((1,), (0,)), ((), ())),
            preferred_element_type=jnp.float32,
        )
        n_ref[0, :, hh, :] = n.astype(jnp.bfloat16)


def _combine_body(n_ref, l_ref, o_ref, ncomm_ref, lcomm_ref, send_sems, recv_sems):
    my_x = lax.axis_index("x")
    my_y = lax.axis_index("y")
    nbr = (1 - my_x, my_y)

    bar = pltpu.get_barrier_semaphore()
    pl.semaphore_signal(
        bar, inc=1, device_id=nbr, device_id_type=pl.DeviceIdType.MESH
    )
    pl.semaphore_wait(bar, 1)

    cp_n = pltpu.make_async_remote_copy(
        src_ref=n_ref, dst_ref=ncomm_ref,
        send_sem=send_sems.at[0], recv_sem=recv_sems.at[0],
        device_id=nbr, device_id_type=pl.DeviceIdType.MESH,
    )
    cp_l = pltpu.make_async_remote_copy(
        src_ref=l_ref, dst_ref=lcomm_ref,
        send_sem=send_sems.at[1], recv_sem=recv_sems.at[1],
        device_id=nbr, device_id_type=pl.DeviceIdType.MESH,
    )
    cp_n.start()
    cp_l.start()
    cp_n.wait()
    cp_l.wait()

    n_sum = n_ref[...].astype(jnp.float32) + ncomm_ref[...].astype(jnp.float32)
    l_sum = l_ref[...] + lcomm_ref[...]
    o_ref[...] = n_sum / l_sum[..., None]


def kernel(Q, K, V):
    b, q, h, d = Q.shape
    kv = K.shape[1]

    n_part, l_part = pl.pallas_call(
        _flash_partial_body,
        grid=(b,),
        in_specs=[
            pl.BlockSpec((1, q, h, d), lambda i: (i, 0, 0, 0)),
            pl.BlockSpec((1, kv, h, d), lambda i: (i, 0, 0, 0)),
            pl.BlockSpec((1, kv, h, d), lambda i: (i, 0, 0, 0)),
        ],
        out_specs=[
            pl.BlockSpec((1, q, h, d), lambda i: (i, 0, 0, 0)),
            pl.BlockSpec((1, q, h), lambda i: (i, 0, 0)),
        ],
        out_shape=[
            jax.ShapeDtypeStruct((b, q, h, d), jnp.bfloat16),
            jax.ShapeDtypeStruct((b, q, h), jnp.float32),
        ],
    )(Q, K, V)

    return pl.pallas_call(
        _combine_body,
        in_specs=[
            pl.BlockSpec(memory_space=pltpu.VMEM),
            pl.BlockSpec(memory_space=pltpu.VMEM),
        ],
        out_specs=pl.BlockSpec(memory_space=pltpu.VMEM),
        out_shape=jax.ShapeDtypeStruct((b, q, h, d), jnp.float32),
        scratch_shapes=[
            pltpu.VMEM((b, q, h, d), jnp.bfloat16),
            pltpu.VMEM((b, q, h), jnp.float32),
            pltpu.SemaphoreType.DMA((2,)),
            pltpu.SemaphoreType.DMA((2,)),
        ],
        compiler_params=pltpu.CompilerParams(collective_id=0),
    )(n_part, l_part)


# device time: 38248 ns/iter; 3.9139x vs baseline; 3.9139x over previous
import functools

import jax
import jax.numpy as jnp
from jax import lax
from jax.experimental import pallas as pl
from jax.experimental.pallas import tpu as pltpu

SCALE = 128 ** -0.5
N_HEADS = 16
HEAD_DIM = 128


def _flash_partial_body(nb_half, q_ref, k_hbm, v_hbm, n_ref, l_ref,
                        kbuf, vbuf, ksems, vsems):
    i = pl.program_id(0)
    my_y = lax.axis_index("y")
    b_idx = my_y * nb_half + i

    copies = []
    for hh in range(N_HEADS):
        ck = pltpu.make_async_copy(
            k_hbm.at[b_idx, :, hh, :], kbuf.at[hh], ksems.at[hh]
        )
        cv = pltpu.make_async_copy(
            v_hbm.at[b_idx, :, hh, :], vbuf.at[hh], vsems.at[hh]
        )
        ck.start()
        cv.start()
        copies.append((ck, cv))

    for hh in range(N_HEADS):
        ck, cv = copies[hh]
        ck.wait()
        cv.wait()
        sl = slice(hh * HEAD_DIM, (hh + 1) * HEAD_DIM)
        q = q_ref[0, :, sl].astype(jnp.bfloat16)
        k = kbuf[hh].astype(jnp.bfloat16)
        v = vbuf[hh].astype(jnp.bfloat16)
        s = lax.dot_general(
            q, k, (((1,), (1,)), ((), ())), preferred_element_type=jnp.float32
        )
        p = jnp.exp(s * SCALE)
        l_ref[0, :, hh] = jnp.sum(p, axis=1)
        n = lax.dot_general(
            p.astype(jnp.bfloat16), v, (((1,), (0,)), ((), ())),
            preferred_element_type=jnp.float32,
        )
        n_ref[0, :, sl] = n.astype(jnp.bfloat16)


def _combine_body(nb_half, n_ref, l_ref, o_ref, ncomm, lcomm,
                  obuf_send, obuf_recv, send_sems, recv_sems):
    my_x = lax.axis_index("x")
    my_y = lax.axis_index("y")
    xn = (1 - my_x, my_y)
    yn = (my_x, 1 - my_y)

    bar = pltpu.get_barrier_semaphore()
    pl.semaphore_signal(bar, inc=1, device_id=xn,
                        device_id_type=pl.DeviceIdType.MESH)
    pl.semaphore_signal(bar, inc=1, device_id=yn,
                        device_id_type=pl.DeviceIdType.MESH)
    pl.semaphore_wait(bar, 2)

    cp_n = pltpu.make_async_remote_copy(
        src_ref=n_ref, dst_ref=ncomm,
        send_sem=send_sems.at[0], recv_sem=recv_sems.at[0],
        device_id=xn, device_id_type=pl.DeviceIdType.MESH,
    )
    cp_l = pltpu.make_async_remote_copy(
        src_ref=l_ref, dst_ref=lcomm,
        send_sem=send_sems.at[1], recv_sem=recv_sems.at[1],
        device_id=xn, device_id_type=pl.DeviceIdType.MESH,
    )
    cp_n.start()
    cp_l.start()
    cp_n.wait()
    cp_l.wait()

    n_sum = n_ref[...].astype(jnp.float32) + ncomm[...].astype(jnp.float32)
    l_sum = l_ref[...] + lcomm[...]
    o_half = n_sum / l_sum[..., None]

    my_b0 = my_y * nb_half
    o_ref[pl.ds(my_b0, nb_half), :, :, :] = o_half
    obuf_send[...] = o_half.astype(jnp.bfloat16)

    cp_o = pltpu.make_async_remote_copy(
        src_ref=obuf_send, dst_ref=obuf_recv,
        send_sem=send_sems.at[2], recv_sem=recv_sems.at[2],
        device_id=yn, device_id_type=pl.DeviceIdType.MESH,
    )
    cp_o.start()
    cp_o.wait()

    other_b0 = (1 - my_y) * nb_half
    o_ref[pl.ds(other_b0, nb_half), :, :, :] = obuf_recv[...].astype(jnp.float32)


def kernel(Q, K, V):
    b, q, h, d = Q.shape
    kv = K.shape[1]
    nb_half = b // 2

    my_y = lax.axis_index("y")
    q_half = lax.dynamic_slice_in_dim(
        Q.reshape(b, q, h * d), my_y * nb_half, nb_half, axis=0
    )

    n_part, l_part = pl.pallas_call(
        functools.partial(_flash_partial_body, nb_half),
        grid=(nb_half,),
        in_specs=[
            pl.BlockSpec((1, q, h * d), lambda i: (i, 0, 0)),
            pl.BlockSpec(memory_space=pl.ANY),
            pl.BlockSpec(memory_space=pl.ANY),
        ],
        out_specs=[
            pl.BlockSpec((1, q, h * d), lambda i: (i, 0, 0)),
            pl.BlockSpec((1, q, h), lambda i: (i, 0, 0)),
        ],
        out_shape=[
            jax.ShapeDtypeStruct((nb_half, q, h * d), jnp.bfloat16),
            jax.ShapeDtypeStruct((nb_half, q, h), jnp.float32),
        ],
        scratch_shapes=[
            pltpu.VMEM((N_HEADS, kv, d), jnp.float32),
            pltpu.VMEM((N_HEADS, kv, d), jnp.float32),
            pltpu.SemaphoreType.DMA((N_HEADS,)),
            pltpu.SemaphoreType.DMA((N_HEADS,)),
        ],
    )(q_half, K, V)
    n_part = n_part.reshape(nb_half, q, h, d)

    return pl.pallas_call(
        functools.partial(_combine_body, nb_half),
        in_specs=[
            pl.BlockSpec(memory_space=pltpu.VMEM),
            pl.BlockSpec(memory_space=pltpu.VMEM),
        ],
        out_specs=pl.BlockSpec(memory_space=pltpu.VMEM),
        out_shape=jax.ShapeDtypeStruct((b, q, h, d), jnp.float32),
        scratch_shapes=[
            pltpu.VMEM((nb_half, q, h, d), jnp.bfloat16),
            pltpu.VMEM((nb_half, q, h), jnp.float32),
            pltpu.VMEM((nb_half, q, h, d), jnp.bfloat16),
            pltpu.VMEM((nb_half, q, h, d), jnp.bfloat16),
            pltpu.SemaphoreType.DMA((3,)),
            pltpu.SemaphoreType.DMA((3,)),
        ],
        compiler_params=pltpu.CompilerParams(collective_id=0),
    )(n_part, l_part)


# device time: 34016 ns/iter; 4.4009x vs baseline; 1.1244x over previous
import functools

import jax
import jax.numpy as jnp
from jax import lax
from jax.experimental import pallas as pl
from jax.experimental.pallas import tpu as pltpu

SCALE = 128 ** -0.5
N_HEADS = 16
HEAD_DIM = 128
D = HEAD_DIM


def _fused_body(nb_half, q_hbm, k_hbm, v_hbm, o_hbm,
                qbuf, qsems, kbuf, vbuf, ksems, vsems,
                nloc, lloc, nrem, lrem, obuf_send, obuf_recv, o_vmem,
                nsend_sems, lsend_sems, nrecv_sems, lrecv_sems,
                osend_sems, orecv_sems, odma_sems):
    i = pl.program_id(0)
    my_x = lax.axis_index("x")
    my_y = lax.axis_index("y")
    xn = (1 - my_x, my_y)
    yn = (my_x, 1 - my_y)
    b_idx = my_y * nb_half + i
    my_b0 = my_y * nb_half
    q_len = q_hbm.shape[1]

    def combine_slot(j):
        pltpu.make_async_remote_copy(
            src_ref=nloc.at[j], dst_ref=nrem.at[j],
            send_sem=nsend_sems.at[j], recv_sem=nrecv_sems.at[j],
            device_id=xn, device_id_type=pl.DeviceIdType.MESH,
        ).wait_recv()
        pltpu.make_async_remote_copy(
            src_ref=lloc.at[j], dst_ref=lrem.at[j],
            send_sem=lsend_sems.at[j], recv_sem=lrecv_sems.at[j],
            device_id=xn, device_id_type=pl.DeviceIdType.MESH,
        ).wait_recv()
        n_sum = nloc[j].astype(jnp.float32) + nrem[j].astype(jnp.float32)
        l_sum = lloc[j] + lrem[j]
        l_bcast = jnp.concatenate(
            [jnp.broadcast_to(l_sum[:, hh][:, None], (q_len, D))
             for hh in range(N_HEADS)],
            axis=1,
        )
        o_flat = n_sum / l_bcast
        obuf_send[j] = o_flat.astype(jnp.bfloat16)
        o_vmem[my_b0 + j, :, :, :] = o_flat.reshape(q_len, N_HEADS, D)
        pltpu.make_async_copy(
            o_vmem.at[my_b0 + j], o_hbm.at[my_b0 + j], odma_sems.at[0, j]
        ).start()
        pltpu.make_async_remote_copy(
            src_ref=obuf_send.at[j], dst_ref=obuf_recv.at[j],
            send_sem=osend_sems.at[j], recv_sem=orecv_sems.at[j],
            device_id=yn, device_id_type=pl.DeviceIdType.MESH,
        ).start()

    @pl.when(i == 0)
    def _():
        bar = pltpu.get_barrier_semaphore()
        pl.semaphore_signal(bar, inc=1, device_id=xn,
                            device_id_type=pl.DeviceIdType.MESH)
        pl.semaphore_signal(bar, inc=1, device_id=yn,
                            device_id_type=pl.DeviceIdType.MESH)
        pl.semaphore_wait(bar, 2)

    def issue_kv(step):
        par = lax.rem(step, 2)
        bb = my_y * nb_half + step
        for hh in range(N_HEADS):
            pltpu.make_async_copy(
                q_hbm.at[bb, :, hh, :], qbuf.at[par, hh], qsems.at[par, hh]
            ).start()
            pltpu.make_async_copy(
                k_hbm.at[bb, :, hh, :], kbuf.at[par, hh], ksems.at[par, hh]
            ).start()
            pltpu.make_async_copy(
                v_hbm.at[bb, :, hh, :], vbuf.at[par, hh], vsems.at[par, hh]
            ).start()

    @pl.when(i == 0)
    def _():
        issue_kv(0)

    @pl.when(i < nb_half - 1)
    def _():
        issue_kv(i + 1)

    par_i = lax.rem(i, 2)
    for hh in range(N_HEADS):
        pltpu.make_async_copy(
            q_hbm.at[b_idx, :, hh, :], qbuf.at[par_i, hh], qsems.at[par_i, hh]
        ).wait()
        pltpu.make_async_copy(
            k_hbm.at[b_idx, :, hh, :], kbuf.at[par_i, hh], ksems.at[par_i, hh]
        ).wait()
        pltpu.make_async_copy(
            v_hbm.at[b_idx, :, hh, :], vbuf.at[par_i, hh], vsems.at[par_i, hh]
        ).wait()
        sl = slice(hh * D, (hh + 1) * D)
        q = qbuf[par_i, hh]
        s = lax.dot_general(
            q, kbuf[par_i, hh], (((1,), (1,)), ((), ())),
            precision=lax.Precision.DEFAULT,
            preferred_element_type=jnp.float32,
        )
        p = jnp.exp(s * SCALE)
        lloc[i, :, hh] = jnp.sum(p, axis=1)
        n = lax.dot_general(
            p, vbuf[par_i, hh], (((1,), (0,)), ((), ())),
            precision=lax.Precision.DEFAULT,
            preferred_element_type=jnp.float32,
        )
        nloc[i, :, sl] = n.astype(jnp.bfloat16)

    cp_n = pltpu.make_async_remote_copy(
        src_ref=nloc.at[i], dst_ref=nrem.at[i],
        send_sem=nsend_sems.at[i], recv_sem=nrecv_sems.at[i],
        device_id=xn, device_id_type=pl.DeviceIdType.MESH,
    )
    cp_l = pltpu.make_async_remote_copy(
        src_ref=lloc.at[i], dst_ref=lrem.at[i],
        send_sem=lsend_sems.at[i], recv_sem=lrecv_sems.at[i],
        device_id=xn, device_id_type=pl.DeviceIdType.MESH,
    )
    cp_n.start()
    cp_l.start()

    for jj in range(nb_half - 1):
        @pl.when(i == jj + 1)
        def _(jj=jj):
            combine_slot(jj)

    @pl.when(i == nb_half - 1)
    def _():
        combine_slot(nb_half - 1)

        other_b0 = (1 - my_y) * nb_half
        for j in range(nb_half):
            pltpu.make_async_remote_copy(
                src_ref=obuf_send.at[j], dst_ref=obuf_recv.at[j],
                send_sem=osend_sems.at[j], recv_sem=orecv_sems.at[j],
                device_id=yn, device_id_type=pl.DeviceIdType.MESH,
            ).wait_recv()
            o_vmem[other_b0 + j, :, :, :] = (
                obuf_recv[j].astype(jnp.float32).reshape(q_len, N_HEADS, D)
            )
            pltpu.make_async_copy(
                o_vmem.at[other_b0 + j], o_hbm.at[other_b0 + j],
                odma_sems.at[1, j],
            ).start()

        for j in range(nb_half):
            pltpu.make_async_copy(
                o_vmem.at[my_b0 + j], o_hbm.at[my_b0 + j], odma_sems.at[0, j]
            ).wait()
            pltpu.make_async_copy(
                o_vmem.at[other_b0 + j], o_hbm.at[other_b0 + j],
                odma_sems.at[1, j],
            ).wait()
        for j in range(nb_half):
            pltpu.make_async_remote_copy(
                src_ref=nloc.at[j], dst_ref=nrem.at[j],
                send_sem=nsend_sems.at[j], recv_sem=nrecv_sems.at[j],
                device_id=xn, device_id_type=pl.DeviceIdType.MESH,
            ).wait_send()
            pltpu.make_async_remote_copy(
                src_ref=lloc.at[j], dst_ref=lrem.at[j],
                send_sem=lsend_sems.at[j], recv_sem=lrecv_sems.at[j],
                device_id=xn, device_id_type=pl.DeviceIdType.MESH,
            ).wait_send()
            pltpu.make_async_remote_copy(
                src_ref=obuf_send.at[j], dst_ref=obuf_recv.at[j],
                send_sem=osend_sems.at[j], recv_sem=orecv_sems.at[j],
                device_id=yn, device_id_type=pl.DeviceIdType.MESH,
            ).wait_send()


def kernel(Q, K, V):
    b, q, h, d = Q.shape
    kv = K.shape[1]
    nb_half = b // 2

    out = pl.pallas_call(
        functools.partial(_fused_body, nb_half),
        grid=(nb_half,),
        in_specs=[
            pl.BlockSpec(memory_space=pl.ANY),
            pl.BlockSpec(memory_space=pl.ANY),
            pl.BlockSpec(memory_space=pl.ANY),
        ],
        out_specs=pl.BlockSpec(memory_space=pl.ANY),
        out_shape=jax.ShapeDtypeStruct((b, q, h, d), jnp.float32),
        scratch_shapes=[
            pltpu.VMEM((2, N_HEADS, q, d), jnp.float32),
            pltpu.SemaphoreType.DMA((2, N_HEADS)),
            pltpu.VMEM((2, N_HEADS, kv, d), jnp.float32),
            pltpu.VMEM((2, N_HEADS, kv, d), jnp.float32),
            pltpu.SemaphoreType.DMA((2, N_HEADS)),
            pltpu.SemaphoreType.DMA((2, N_HEADS)),
            pltpu.VMEM((nb_half, q, h * d), jnp.bfloat16),
            pltpu.VMEM((nb_half, q, h), jnp.float32),
            pltpu.VMEM((nb_half, q, h * d), jnp.bfloat16),
            pltpu.VMEM((nb_half, q, h), jnp.float32),
            pltpu.VMEM((nb_half, q, h * d), jnp.bfloat16),
            pltpu.VMEM((nb_half, q, h * d), jnp.bfloat16),
            pltpu.VMEM((b, q, h, d), jnp.float32),
            pltpu.SemaphoreType.DMA((nb_half,)),
            pltpu.SemaphoreType.DMA((nb_half,)),
            pltpu.SemaphoreType.DMA((nb_half,)),
            pltpu.SemaphoreType.DMA((nb_half,)),
            pltpu.SemaphoreType.DMA((nb_half,)),
            pltpu.SemaphoreType.DMA((nb_half,)),
            pltpu.SemaphoreType.DMA((2, nb_half)),
        ],
        compiler_params=pltpu.CompilerParams(
            collective_id=0, vmem_limit_bytes=56 * 1024 * 1024
        ),
    )(Q, K, V)
    return pltpu.with_memory_space_constraint(out, pltpu.MemorySpace.HBM)


# device time: 33840 ns/iter; 4.4238x vs baseline; 1.0052x over previous
import functools

import jax
import jax.numpy as jnp
from jax import lax
from jax.experimental import pallas as pl
from jax.experimental.pallas import tpu as pltpu

SCALE = 128 ** -0.5
N_HEADS = 16
HEAD_DIM = 128
D = HEAD_DIM


def _fused_body(nb_half, q_hbm, k_hbm, v_hbm, o_hbm,
                qbuf, qsems, kbuf, vbuf, ksems, vsems,
                nloc, lloc, nrem, lrem, obuf_send, obuf_recv, o_vmem,
                nsend_sems, lsend_sems, nrecv_sems, lrecv_sems,
                osend_sems, orecv_sems, odma_sems):
    i = pl.program_id(0)
    my_x = lax.axis_index("x")
    my_y = lax.axis_index("y")
    xn = (1 - my_x, my_y)
    yn = (my_x, 1 - my_y)
    b_idx = my_y * nb_half + i
    my_b0 = my_y * nb_half
    q_len = q_hbm.shape[1]

    def combine_slot(j):
        pltpu.make_async_remote_copy(
            src_ref=nloc.at[j], dst_ref=nrem.at[j],
            send_sem=nsend_sems.at[j], recv_sem=nrecv_sems.at[j],
            device_id=xn, device_id_type=pl.DeviceIdType.MESH,
        ).wait_recv()
        pltpu.make_async_remote_copy(
            src_ref=lloc.at[j], dst_ref=lrem.at[j],
            send_sem=lsend_sems.at[j], recv_sem=lrecv_sems.at[j],
            device_id=xn, device_id_type=pl.DeviceIdType.MESH,
        ).wait_recv()
        n_sum = nloc[j].astype(jnp.float32) + nrem[j].astype(jnp.float32)
        l_sum = lloc[j] + lrem[j]
        l_bcast = jnp.concatenate(
            [jnp.broadcast_to(l_sum[:, hh][:, None], (q_len, D))
             for hh in range(N_HEADS)],
            axis=1,
        )
        o_flat = n_sum / l_bcast
        obuf_send[j] = o_flat.astype(jnp.bfloat16)
        o_vmem[my_b0 + j, :, :, :] = o_flat.reshape(q_len, N_HEADS, D)
        pltpu.make_async_copy(
            o_vmem.at[my_b0 + j], o_hbm.at[my_b0 + j], odma_sems.at[0, j]
        ).start()
        pltpu.make_async_remote_copy(
            src_ref=obuf_send.at[j], dst_ref=obuf_recv.at[j],
            send_sem=osend_sems.at[j], recv_sem=orecv_sems.at[j],
            device_id=yn, device_id_type=pl.DeviceIdType.MESH,
        ).start()

    def issue_kv(step):
        par = lax.rem(step, 2)
        bb = my_y * nb_half + step
        for hh in range(N_HEADS):
            pltpu.make_async_copy(
                q_hbm.at[bb, :, hh, :], qbuf.at[par, hh], qsems.at[par, hh]
            ).start()
            pltpu.make_async_copy(
                k_hbm.at[bb, :, hh, :], kbuf.at[par, hh], ksems.at[par, hh]
            ).start()
            pltpu.make_async_copy(
                v_hbm.at[bb, :, hh, :], vbuf.at[par, hh], vsems.at[par, hh]
            ).start()

    @pl.when(i == 0)
    def _():
        issue_kv(0)

    @pl.when(i < nb_half - 1)
    def _():
        issue_kv(i + 1)

    @pl.when(i == 0)
    def _():
        bar = pltpu.get_barrier_semaphore()
        pl.semaphore_signal(bar, inc=1, device_id=xn,
                            device_id_type=pl.DeviceIdType.MESH)
        pl.semaphore_signal(bar, inc=1, device_id=yn,
                            device_id_type=pl.DeviceIdType.MESH)
        pl.semaphore_wait(bar, 2)

    par_i = lax.rem(i, 2)
    for hh in range(N_HEADS):
        pltpu.make_async_copy(
            q_hbm.at[b_idx, :, hh, :], qbuf.at[par_i, hh], qsems.at[par_i, hh]
        ).wait()
        pltpu.make_async_copy(
            k_hbm.at[b_idx, :, hh, :], kbuf.at[par_i, hh], ksems.at[par_i, hh]
        ).wait()
        pltpu.make_async_copy(
            v_hbm.at[b_idx, :, hh, :], vbuf.at[par_i, hh], vsems.at[par_i, hh]
        ).wait()
        sl = slice(hh * D, (hh + 1) * D)
        q = qbuf[par_i, hh]
        s = lax.dot_general(
            q, kbuf[par_i, hh], (((1,), (1,)), ((), ())),
            precision=lax.Precision.DEFAULT,
            preferred_element_type=jnp.float32,
        )
        p = jnp.exp(s * SCALE)
        lloc[i, :, hh] = jnp.sum(p, axis=1)
        n = lax.dot_general(
            p, vbuf[par_i, hh], (((1,), (0,)), ((), ())),
            precision=lax.Precision.DEFAULT,
            preferred_element_type=jnp.float32,
        )
        nloc[i, :, sl] = n.astype(jnp.bfloat16)

    cp_n = pltpu.make_async_remote_copy(
        src_ref=nloc.at[i], dst_ref=nrem.at[i],
        send_sem=nsend_sems.at[i], recv_sem=nrecv_sems.at[i],
        device_id=xn, device_id_type=pl.DeviceIdType.MESH,
    )
    cp_l = pltpu.make_async_remote_copy(
        src_ref=lloc.at[i], dst_ref=lrem.at[i],
        send_sem=lsend_sems.at[i], recv_sem=lrecv_sems.at[i],
        device_id=xn, device_id_type=pl.DeviceIdType.MESH,
    )
    cp_n.start()
    cp_l.start()

    for jj in range(nb_half - 1):
        @pl.when(i == jj + 1)
        def _(jj=jj):
            combine_slot(jj)

    @pl.when(i == nb_half - 1)
    def _():
        combine_slot(nb_half - 1)

        other_b0 = (1 - my_y) * nb_half
        for j in range(nb_half):
            pltpu.make_async_remote_copy(
                src_ref=obuf_send.at[j], dst_ref=obuf_recv.at[j],
                send_sem=osend_sems.at[j], recv_sem=orecv_sems.at[j],
                device_id=yn, device_id_type=pl.DeviceIdType.MESH,
            ).wait_recv()
            o_vmem[other_b0 + j, :, :, :] = (
                obuf_recv[j].astype(jnp.float32).reshape(q_len, N_HEADS, D)
            )
            pltpu.make_async_copy(
                o_vmem.at[other_b0 + j], o_hbm.at[other_b0 + j],
                odma_sems.at[1, j],
            ).start()

        for j in range(nb_half):
            pltpu.make_async_copy(
                o_vmem.at[my_b0 + j], o_hbm.at[my_b0 + j], odma_sems.at[0, j]
            ).wait()
            pltpu.make_async_copy(
                o_vmem.at[other_b0 + j], o_hbm.at[other_b0 + j],
                odma_sems.at[1, j],
            ).wait()
        for j in range(nb_half):
            pltpu.make_async_remote_copy(
                src_ref=nloc.at[j], dst_ref=nrem.at[j],
                send_sem=nsend_sems.at[j], recv_sem=nrecv_sems.at[j],
                device_id=xn, device_id_type=pl.DeviceIdType.MESH,
            ).wait_send()
            pltpu.make_async_remote_copy(
                src_ref=lloc.at[j], dst_ref=lrem.at[j],
                send_sem=lsend_sems.at[j], recv_sem=lrecv_sems.at[j],
                device_id=xn, device_id_type=pl.DeviceIdType.MESH,
            ).wait_send()
            pltpu.make_async_remote_copy(
                src_ref=obuf_send.at[j], dst_ref=obuf_recv.at[j],
                send_sem=osend_sems.at[j], recv_sem=orecv_sems.at[j],
                device_id=yn, device_id_type=pl.DeviceIdType.MESH,
            ).wait_send()


def kernel(Q, K, V):
    b, q, h, d = Q.shape
    kv = K.shape[1]
    nb_half = b // 2

    out = pl.pallas_call(
        functools.partial(_fused_body, nb_half),
        grid=(nb_half,),
        in_specs=[
            pl.BlockSpec(memory_space=pl.ANY),
            pl.BlockSpec(memory_space=pl.ANY),
            pl.BlockSpec(memory_space=pl.ANY),
        ],
        out_specs=pl.BlockSpec(memory_space=pltpu.MemorySpace.HBM),
        out_shape=jax.ShapeDtypeStruct((b, q, h, d), jnp.float32),
        scratch_shapes=[
            pltpu.VMEM((2, N_HEADS, q, d), jnp.float32),
            pltpu.SemaphoreType.DMA((2, N_HEADS)),
            pltpu.VMEM((2, N_HEADS, kv, d), jnp.float32),
            pltpu.VMEM((2, N_HEADS, kv, d), jnp.float32),
            pltpu.SemaphoreType.DMA((2, N_HEADS)),
            pltpu.SemaphoreType.DMA((2, N_HEADS)),
            pltpu.VMEM((nb_half, q, h * d), jnp.bfloat16),
            pltpu.VMEM((nb_half, q, h), jnp.float32),
            pltpu.VMEM((nb_half, q, h * d), jnp.bfloat16),
            pltpu.VMEM((nb_half, q, h), jnp.float32),
            pltpu.VMEM((nb_half, q, h * d), jnp.bfloat16),
            pltpu.VMEM((nb_half, q, h * d), jnp.bfloat16),
            pltpu.VMEM((b, q, h, d), jnp.float32),
            pltpu.SemaphoreType.DMA((nb_half,)),
            pltpu.SemaphoreType.DMA((nb_half,)),
            pltpu.SemaphoreType.DMA((nb_half,)),
            pltpu.SemaphoreType.DMA((nb_half,)),
            pltpu.SemaphoreType.DMA((nb_half,)),
            pltpu.SemaphoreType.DMA((nb_half,)),
            pltpu.SemaphoreType.DMA((2, nb_half)),
        ],
        compiler_params=pltpu.CompilerParams(
            collective_id=0, vmem_limit_bytes=56 * 1024 * 1024
        ),
    )(Q, K, V)
    return pltpu.with_memory_space_constraint(out, pltpu.MemorySpace.HBM)
